# baseline (speedup 1.0000x reference)
import functools

import jax
import jax.numpy as jnp
from jax import lax
from jax.experimental import pallas as pl
from jax.experimental.pallas import tpu as pltpu
from jax.experimental.pallas import tpu_sc as plsc

_GRID = 128 ** 3
_N = _GRID // 4
_DECAY = 0.95

_NC = 2
_NS = 16
_NW = _NC * _NS
_CELLS = _GRID // _NW
_W = 8192
_NWIN = _N // _W
_SW = 4096
_NSW = _CELLS // _SW


def _sc_body(mem_hbm, idx_hbm, val_hbm, out_hbm,
             tmp, idx0, val0, idx1, val1, mem0, mem1, out0, out1,
             si0, si1, sm0, sm1, so0, so1):
    wid = lax.axis_index("c") * _NS + lax.axis_index("s")
    base = wid * _CELLS

    pltpu.async_copy(idx_hbm.at[pl.ds(0, _W)], idx0, si0)
    pltpu.async_copy(val_hbm.at[pl.ds(0, _W)], val0, si0)
    pltpu.async_copy(idx_hbm.at[pl.ds(_W, _W)], idx1, si1)
    pltpu.async_copy(val_hbm.at[pl.ds(_W, _W)], val1, si1)

    @functools.partial(lax.fori_loop, 0, _CELLS // 16, unroll=8, init_val=0)
    def _init(i, c):
        tmp[pl.ds(pl.multiple_of(i * 16, 16), 16)] = jnp.full((16,), -1.0, jnp.float32)
        return c

    def _scan_outer(wo, c):
        for b, (ib, vb, sem) in enumerate(((idx0, val0, si0), (idx1, val1, si1))):
            w = 2 * wo + b
            pltpu.make_async_copy(idx_hbm.at[pl.ds(0, _W)], ib, sem).wait()
            pltpu.make_async_copy(val_hbm.at[pl.ds(0, _W)], vb, sem).wait()

            def _scan_vec(j, c2, ib=ib, vb=vb):
                off = pl.multiple_of(j * 16, 16)
                iv = ib[pl.ds(off, 16)]
                vv = vb[pl.ds(off, 16)]
                loc = iv - base
                msk = plsc.bitcast(loc, jnp.uint32) < jnp.uint32(_CELLS)
                plsc.store_scatter(tmp, [loc], vv, mask=msk)
                return c2

            c = lax.fori_loop(0, _W // 16, _scan_vec, c, unroll=16)

            @pl.when(w + 2 < _NWIN)
            def _prefetch(ib=ib, vb=vb, sem=sem, w=w):
                noff = pl.multiple_of((w + 2) * _W, _W)
                pltpu.async_copy(idx_hbm.at[pl.ds(noff, _W)], ib, sem)
                pltpu.async_copy(val_hbm.at[pl.ds(noff, _W)], vb, sem)
        return c

    lax.fori_loop(0, _NWIN // 2, _scan_outer, 0)

    pltpu.async_copy(mem_hbm.at[pl.ds(base, _SW)], mem0, sm0)
    pltpu.async_copy(mem_hbm.at[pl.ds(base + _SW, _SW)], mem1, sm1)

    def _sweep_outer(so, c):
        for b, (mb, ob, smem, sout) in enumerate(
                ((mem0, out0, sm0, so0), (mem1, out1, sm1, so1))):
            s = 2 * so + b
            soff = pl.multiple_of(s * _SW, _SW)
            pltpu.make_async_copy(mem_hbm.at[pl.ds(0, _SW)], mb, smem).wait()

            @pl.when(s >= 2)
            def _wait_out(ob=ob, sout=sout):
                pltpu.make_async_copy(ob, out_hbm.at[pl.ds(0, _SW)], sout).wait()

            def _merge_vec(j, c2, mb=mb, ob=ob, soff=soff):
                off = pl.multiple_of(j * 16, 16)
                t = tmp[pl.ds(soff + off, 16)]
                m = mb[pl.ds(off, 16)]
                ob[pl.ds(off, 16)] = jnp.where(
                    (m >= 0) & (t >= 0), jnp.maximum(m * _DECAY, t), m)
                return c2

            c = lax.fori_loop(0, _SW // 16, _merge_vec, c, unroll=16)
            pltpu.async_copy(ob, out_hbm.at[pl.ds(base + soff, _SW)], sout)

            @pl.when(s + 2 < _NSW)
            def _prefetch_mem(mb=mb, smem=smem, s=s):
                noff = pl.multiple_of((s + 2) * _SW, _SW)
                pltpu.async_copy(mem_hbm.at[pl.ds(base + noff, _SW)], mb, smem)
        return c

    lax.fori_loop(0, _NSW // 2, _sweep_outer, 0)

    pltpu.make_async_copy(out0, out_hbm.at[pl.ds(0, _SW)], so0).wait()
    pltpu.make_async_copy(out1, out_hbm.at[pl.ds(0, _SW)], so1).wait()


@jax.jit
def _run(mem, idx, val):
    mesh = plsc.VectorSubcoreMesh(
        core_axis_name="c", subcore_axis_name="s", num_cores=_NC, num_subcores=_NS)
    return pl.kernel(
        _sc_body,
        out_type=jax.ShapeDtypeStruct((_GRID,), jnp.float32),
        mesh=mesh,
        compiler_params=pltpu.CompilerParams(needs_layout_passes=False),
        scratch_types=[
            pltpu.VMEM((_CELLS,), jnp.float32),
            pltpu.VMEM((_W,), jnp.int32),
            pltpu.VMEM((_W,), jnp.float32),
            pltpu.VMEM((_W,), jnp.int32),
            pltpu.VMEM((_W,), jnp.float32),
            pltpu.VMEM((_SW,), jnp.float32),
            pltpu.VMEM((_SW,), jnp.float32),
            pltpu.VMEM((_SW,), jnp.float32),
            pltpu.VMEM((_SW,), jnp.float32),
            pltpu.SemaphoreType.DMA,
            pltpu.SemaphoreType.DMA,
            pltpu.SemaphoreType.DMA,
            pltpu.SemaphoreType.DMA,
            pltpu.SemaphoreType.DMA,
            pltpu.SemaphoreType.DMA,
        ],
    )(mem, idx, val)


def kernel(mem, idx, val):
    return _run(mem, idx.astype(jnp.int32), val)


# speedup vs baseline: 2.1350x; 2.1350x over previous
import functools

import jax
import jax.numpy as jnp
from jax import lax
from jax.experimental import pallas as pl
from jax.experimental.pallas import tpu as pltpu
from jax.experimental.pallas import tpu_sc as plsc

_GRID = 128 ** 3
_N = _GRID // 4
_DECAY = 0.95

_NC = 2
_NS = 16
_NW = _NC * _NS
_CELLS = _GRID // _NW
_W = 8192
_NWIN = _N // _W
_SW = 4096
_NSW = _CELLS // _SW


def _sc_body(mem_hbm, idx_hbm, val_hbm, out_hbm,
             tmp, idx0, val0, idx1, val1, mem0, mem1, out0, out1,
             si0, si1, sm0, sm1, so0, so1):
    wid = lax.axis_index("c") * _NS + lax.axis_index("s")
    base = wid * _CELLS

    pltpu.async_copy(idx_hbm.at[pl.ds(0, _W)], idx0, si0)
    pltpu.async_copy(val_hbm.at[pl.ds(0, _W)], val0, si0)
    pltpu.async_copy(idx_hbm.at[pl.ds(_W, _W)], idx1, si1)
    pltpu.async_copy(val_hbm.at[pl.ds(_W, _W)], val1, si1)

    @functools.partial(lax.fori_loop, 0, _CELLS // 16, unroll=8, init_val=0)
    def _init(i, c):
        tmp[pl.ds(pl.multiple_of(i * 16, 16), 16)] = jnp.full((16,), -1.0, jnp.float32)
        return c

    def _scan_outer(wo, c):
        for b, (ib, vb, sem) in enumerate(((idx0, val0, si0), (idx1, val1, si1))):
            w = 2 * wo + b
            pltpu.make_async_copy(idx_hbm.at[pl.ds(0, _W)], ib, sem).wait()
            pltpu.make_async_copy(val_hbm.at[pl.ds(0, _W)], vb, sem).wait()

            def _scan_blk(jb, c2, ib=ib, vb=vb):
                offb = pl.multiple_of(jb * 128, 128)
                pairs = []
                for u in range(8):
                    off = offb + u * 16
                    pairs.append((ib[pl.ds(off, 16)], vb[pl.ds(off, 16)]))
                for iv, vv in pairs:
                    loc = iv - base
                    msk = plsc.bitcast(loc, jnp.uint32) < jnp.uint32(_CELLS)
                    plsc.store_scatter(tmp, [loc], vv, mask=msk)
                return c2

            c = lax.fori_loop(0, _W // 128, _scan_blk, c, unroll=4)

            @pl.when(w + 2 < _NWIN)
            def _prefetch(ib=ib, vb=vb, sem=sem, w=w):
                noff = pl.multiple_of((w + 2) * _W, _W)
                pltpu.async_copy(idx_hbm.at[pl.ds(noff, _W)], ib, sem)
                pltpu.async_copy(val_hbm.at[pl.ds(noff, _W)], vb, sem)
        return c

    lax.fori_loop(0, _NWIN // 2, _scan_outer, 0)

    pltpu.async_copy(mem_hbm.at[pl.ds(base, _SW)], mem0, sm0)
    pltpu.async_copy(mem_hbm.at[pl.ds(base + _SW, _SW)], mem1, sm1)

    def _sweep_outer(so, c):
        for b, (mb, ob, smem, sout) in enumerate(
                ((mem0, out0, sm0, so0), (mem1, out1, sm1, so1))):
            s = 2 * so + b
            soff = pl.multiple_of(s * _SW, _SW)
            pltpu.make_async_copy(mem_hbm.at[pl.ds(0, _SW)], mb, smem).wait()

            @pl.when(s >= 2)
            def _wait_out(ob=ob, sout=sout):
                pltpu.make_async_copy(ob, out_hbm.at[pl.ds(0, _SW)], sout).wait()

            def _merge_vec(j, c2, mb=mb, ob=ob, soff=soff):
                off = pl.multiple_of(j * 16, 16)
                t = tmp[pl.ds(soff + off, 16)]
                m = mb[pl.ds(off, 16)]
                ob[pl.ds(off, 16)] = jnp.where(
                    (m >= 0) & (t >= 0), jnp.maximum(m * _DECAY, t), m)
                return c2

            c = lax.fori_loop(0, _SW // 16, _merge_vec, c, unroll=16)
            pltpu.async_copy(ob, out_hbm.at[pl.ds(base + soff, _SW)], sout)

            @pl.when(s + 2 < _NSW)
            def _prefetch_mem(mb=mb, smem=smem, s=s):
                noff = pl.multiple_of((s + 2) * _SW, _SW)
                pltpu.async_copy(mem_hbm.at[pl.ds(base + noff, _SW)], mb, smem)
        return c

    lax.fori_loop(0, _NSW // 2, _sweep_outer, 0)

    pltpu.make_async_copy(out0, out_hbm.at[pl.ds(0, _SW)], so0).wait()
    pltpu.make_async_copy(out1, out_hbm.at[pl.ds(0, _SW)], so1).wait()


@jax.jit
def _run(mem, idx, val):
    mesh = plsc.VectorSubcoreMesh(
        core_axis_name="c", subcore_axis_name="s", num_cores=_NC, num_subcores=_NS)
    return pl.kernel(
        _sc_body,
        out_type=jax.ShapeDtypeStruct((_GRID,), jnp.float32),
        mesh=mesh,
        compiler_params=pltpu.CompilerParams(needs_layout_passes=False),
        scratch_types=[
            pltpu.VMEM((_CELLS,), jnp.float32),
            pltpu.VMEM((_W,), jnp.int32),
            pltpu.VMEM((_W,), jnp.float32),
            pltpu.VMEM((_W,), jnp.int32),
            pltpu.VMEM((_W,), jnp.float32),
            pltpu.VMEM((_SW,), jnp.float32),
            pltpu.VMEM((_SW,), jnp.float32),
            pltpu.VMEM((_SW,), jnp.float32),
            pltpu.VMEM((_SW,), jnp.float32),
            pltpu.SemaphoreType.DMA,
            pltpu.SemaphoreType.DMA,
            pltpu.SemaphoreType.DMA,
            pltpu.SemaphoreType.DMA,
            pltpu.SemaphoreType.DMA,
            pltpu.SemaphoreType.DMA,
        ],
    )(mem, idx, val)


def kernel(mem, idx, val):
    return _run(mem, idx.astype(jnp.int32), val)


# speedup vs baseline: 2.1744x; 1.0185x over previous
import functools

import jax
import jax.numpy as jnp
from jax import lax
from jax.experimental import pallas as pl
from jax.experimental.pallas import tpu as pltpu
from jax.experimental.pallas import tpu_sc as plsc

_GRID = 128 ** 3
_N = _GRID // 4
_DECAY = 0.95

_NC = 2
_NS = 16
_NW = _NC * _NS
_CELLS = _GRID // _NW
_W = 8192
_NWIN = _N // _W
_SW = 4096
_NSW = _CELLS // _SW


def _sc_body(mem_hbm, idx_hbm, val_hbm, out_hbm,
             tmp, idx0, val0, idx1, val1, mem0, mem1, out0, out1,
             si0, si1, sm0, sm1, so0, so1):
    wid = lax.axis_index("c") * _NS + lax.axis_index("s")
    base = wid * _CELLS

    pltpu.async_copy(idx_hbm.at[pl.ds(0, _W)], idx0, si0)
    pltpu.async_copy(val_hbm.at[pl.ds(0, _W)], val0, si0)
    pltpu.async_copy(idx_hbm.at[pl.ds(_W, _W)], idx1, si1)
    pltpu.async_copy(val_hbm.at[pl.ds(_W, _W)], val1, si1)

    @functools.partial(lax.fori_loop, 0, _CELLS // 16, unroll=8, init_val=0)
    def _init(i, c):
        tmp[pl.ds(pl.multiple_of(i * 16, 16), 16)] = jnp.full((16,), -1.0, jnp.float32)
        return c

    def _scan_outer(wo, c):
        for b, (ib, vb, sem) in enumerate(((idx0, val0, si0), (idx1, val1, si1))):
            w = 2 * wo + b
            pltpu.make_async_copy(idx_hbm.at[pl.ds(0, _W)], ib, sem).wait()
            pltpu.make_async_copy(val_hbm.at[pl.ds(0, _W)], vb, sem).wait()

            def _scan_blk(jb, c2, ib=ib, vb=vb):
                offb = pl.multiple_of(jb * 256, 256)
                pairs = []
                for u in range(16):
                    off = offb + u * 16
                    pairs.append((ib[pl.ds(off, 16)], vb[pl.ds(off, 16)]))
                for iv, vv in pairs:
                    loc = iv - base
                    msk = plsc.bitcast(loc, jnp.uint32) < jnp.uint32(_CELLS)
                    plsc.store_scatter(tmp, [loc], vv, mask=msk)
                return c2

            c = lax.fori_loop(0, _W // 256, _scan_blk, c, unroll=2)

            @pl.when(w + 2 < _NWIN)
            def _prefetch(ib=ib, vb=vb, sem=sem, w=w):
                noff = pl.multiple_of((w + 2) * _W, _W)
                pltpu.async_copy(idx_hbm.at[pl.ds(noff, _W)], ib, sem)
                pltpu.async_copy(val_hbm.at[pl.ds(noff, _W)], vb, sem)
        return c

    lax.fori_loop(0, _NWIN // 2, _scan_outer, 0)

    pltpu.async_copy(mem_hbm.at[pl.ds(base, _SW)], mem0, sm0)
    pltpu.async_copy(mem_hbm.at[pl.ds(base + _SW, _SW)], mem1, sm1)

    def _sweep_outer(so, c):
        for b, (mb, ob, smem, sout) in enumerate(
                ((mem0, out0, sm0, so0), (mem1, out1, sm1, so1))):
            s = 2 * so + b
            soff = pl.multiple_of(s * _SW, _SW)
            pltpu.make_async_copy(mem_hbm.at[pl.ds(0, _SW)], mb, smem).wait()

            @pl.when(s >= 2)
            def _wait_out(ob=ob, sout=sout):
                pltpu.make_async_copy(ob, out_hbm.at[pl.ds(0, _SW)], sout).wait()

            def _merge_vec(j, c2, mb=mb, ob=ob, soff=soff):
                off = pl.multiple_of(j * 16, 16)
                t = tmp[pl.ds(soff + off, 16)]
                m = mb[pl.ds(off, 16)]
                ob[pl.ds(off, 16)] = jnp.where(
                    (m >= 0) & (t >= 0), jnp.maximum(m * _DECAY, t), m)
                return c2

            c = lax.fori_loop(0, _SW // 16, _merge_vec, c, unroll=16)
            pltpu.async_copy(ob, out_hbm.at[pl.ds(base + soff, _SW)], sout)

            @pl.when(s + 2 < _NSW)
            def _prefetch_mem(mb=mb, smem=smem, s=s):
                noff = pl.multiple_of((s + 2) * _SW, _SW)
                pltpu.async_copy(mem_hbm.at[pl.ds(base + noff, _SW)], mb, smem)
        return c

    lax.fori_loop(0, _NSW // 2, _sweep_outer, 0)

    pltpu.make_async_copy(out0, out_hbm.at[pl.ds(0, _SW)], so0).wait()
    pltpu.make_async_copy(out1, out_hbm.at[pl.ds(0, _SW)], so1).wait()


@jax.jit
def _run(mem, idx, val):
    mesh = plsc.VectorSubcoreMesh(
        core_axis_name="c", subcore_axis_name="s", num_cores=_NC, num_subcores=_NS)
    return pl.kernel(
        _sc_body,
        out_type=jax.ShapeDtypeStruct((_GRID,), jnp.float32),
        mesh=mesh,
        compiler_params=pltpu.CompilerParams(needs_layout_passes=False),
        scratch_types=[
            pltpu.VMEM((_CELLS,), jnp.float32),
            pltpu.VMEM((_W,), jnp.int32),
            pltpu.VMEM((_W,), jnp.float32),
            pltpu.VMEM((_W,), jnp.int32),
            pltpu.VMEM((_W,), jnp.float32),
            pltpu.VMEM((_SW,), jnp.float32),
            pltpu.VMEM((_SW,), jnp.float32),
            pltpu.VMEM((_SW,), jnp.float32),
            pltpu.VMEM((_SW,), jnp.float32),
            pltpu.SemaphoreType.DMA,
            pltpu.SemaphoreType.DMA,
            pltpu.SemaphoreType.DMA,
            pltpu.SemaphoreType.DMA,
            pltpu.SemaphoreType.DMA,
            pltpu.SemaphoreType.DMA,
        ],
    )(mem, idx, val)


def kernel(mem, idx, val):
    return _run(mem, idx.astype(jnp.int32), val)


# speedup vs baseline: 2.3919x; 1.1000x over previous
import functools

import jax
import jax.numpy as jnp
from jax import lax
from jax.experimental import pallas as pl
from jax.experimental.pallas import tpu as pltpu
from jax.experimental.pallas import tpu_sc as plsc

_GRID = 128 ** 3
_N = _GRID // 4
_DECAY = 0.95

_NC = 2
_NS = 16
_NW = _NC * _NS
_CELLS = _GRID // _NW
_W = 8192
_NWIN = _N // _W
_SW = 4096
_NSW = _CELLS // _SW


def _sc_body(mem_hbm, idx_hbm, val_hbm, out_hbm,
             tmp, idx0, val0, idx1, val1, mem0, mem1, out0, out1,
             si0, si1, sm0, sm1, so0, so1):
    wid = lax.axis_index("c") * _NS + lax.axis_index("s")
    base = wid * _CELLS

    pltpu.async_copy(idx_hbm.at[pl.ds(0, _W)], idx0, si0)
    pltpu.async_copy(val_hbm.at[pl.ds(0, _W)], val0, si0)
    pltpu.async_copy(idx_hbm.at[pl.ds(_W, _W)], idx1, si1)
    pltpu.async_copy(val_hbm.at[pl.ds(_W, _W)], val1, si1)

    @functools.partial(lax.fori_loop, 0, _CELLS // 16, unroll=8, init_val=0)
    def _init(i, c):
        tmp[pl.ds(pl.multiple_of(i * 16, 16), 16)] = jnp.full((16,), -1.0, jnp.float32)
        return c

    def _scan_outer(wo, c):
        for b, (ib, vb, sem) in enumerate(((idx0, val0, si0), (idx1, val1, si1))):
            w = 2 * wo + b
            pltpu.make_async_copy(idx_hbm.at[pl.ds(0, _W)], ib, sem).wait()
            pltpu.make_async_copy(val_hbm.at[pl.ds(0, _W)], vb, sem).wait()

            def _scan_blk(jb, c2, ib=ib, vb=vb):
                offb = pl.multiple_of(jb * 256, 256)
                pairs = []
                for u in range(16):
                    off = offb + u * 16
                    pairs.append((ib[pl.ds(off, 16)], vb[pl.ds(off, 16)]))
                for iv, vv in pairs:
                    loc = iv - base
                    msk = plsc.bitcast(loc, jnp.uint32) < jnp.uint32(_CELLS)
                    plsc.store_scatter(tmp, [loc], vv, mask=msk)
                return c2

            c = lax.fori_loop(0, _W // 256, _scan_blk, c, unroll=2)

            @pl.when(w + 2 < _NWIN)
            def _prefetch(ib=ib, vb=vb, sem=sem, w=w):
                noff = pl.multiple_of((w + 2) * _W, _W)
                pltpu.async_copy(idx_hbm.at[pl.ds(noff, _W)], ib, sem)
                pltpu.async_copy(val_hbm.at[pl.ds(noff, _W)], vb, sem)
        return c

    lax.fori_loop(0, _NWIN // 2, _scan_outer, 0)

    pltpu.async_copy(mem_hbm.at[pl.ds(base, _SW)], mem0, sm0)
    pltpu.async_copy(mem_hbm.at[pl.ds(base + _SW, _SW)], mem1, sm1)

    def _sweep_outer(so, c):
        for b, (mb, ob, smem, sout) in enumerate(
                ((mem0, out0, sm0, so0), (mem1, out1, sm1, so1))):
            s = 2 * so + b
            soff = pl.multiple_of(s * _SW, _SW)
            pltpu.make_async_copy(mem_hbm.at[pl.ds(0, _SW)], mb, smem).wait()

            @pl.when(s >= 2)
            def _wait_out(ob=ob, sout=sout):
                pltpu.make_async_copy(ob, out_hbm.at[pl.ds(0, _SW)], sout).wait()

            def _merge_blk(j, c2, mb=mb, ob=ob, soff=soff):
                offb = pl.multiple_of(j * 128, 128)
                pairs = []
                for u in range(8):
                    off = offb + u * 16
                    pairs.append((tmp[pl.ds(soff + off, 16)], mb[pl.ds(off, 16)]))
                for u, (t, m) in enumerate(pairs):
                    ob[pl.ds(offb + u * 16, 16)] = jnp.where(
                        (m >= 0) & (t >= 0), jnp.maximum(m * _DECAY, t), m)
                return c2

            c = lax.fori_loop(0, _SW // 128, _merge_blk, c, unroll=4)
            pltpu.async_copy(ob, out_hbm.at[pl.ds(base + soff, _SW)], sout)

            @pl.when(s + 2 < _NSW)
            def _prefetch_mem(mb=mb, smem=smem, s=s):
                noff = pl.multiple_of((s + 2) * _SW, _SW)
                pltpu.async_copy(mem_hbm.at[pl.ds(base + noff, _SW)], mb, smem)
        return c

    lax.fori_loop(0, _NSW // 2, _sweep_outer, 0)

    pltpu.make_async_copy(out0, out_hbm.at[pl.ds(0, _SW)], so0).wait()
    pltpu.make_async_copy(out1, out_hbm.at[pl.ds(0, _SW)], so1).wait()


@jax.jit
def _run(mem, idx, val):
    mesh = plsc.VectorSubcoreMesh(
        core_axis_name="c", subcore_axis_name="s", num_cores=_NC, num_subcores=_NS)
    return pl.kernel(
        _sc_body,
        out_type=jax.ShapeDtypeStruct((_GRID,), jnp.float32),
        mesh=mesh,
        compiler_params=pltpu.CompilerParams(needs_layout_passes=False),
        scratch_types=[
            pltpu.VMEM((_CELLS,), jnp.float32),
            pltpu.VMEM((_W,), jnp.int32),
            pltpu.VMEM((_W,), jnp.float32),
            pltpu.VMEM((_W,), jnp.int32),
            pltpu.VMEM((_W,), jnp.float32),
            pltpu.VMEM((_SW,), jnp.float32),
            pltpu.VMEM((_SW,), jnp.float32),
            pltpu.VMEM((_SW,), jnp.float32),
            pltpu.VMEM((_SW,), jnp.float32),
            pltpu.SemaphoreType.DMA,
            pltpu.SemaphoreType.DMA,
            pltpu.SemaphoreType.DMA,
            pltpu.SemaphoreType.DMA,
            pltpu.SemaphoreType.DMA,
            pltpu.SemaphoreType.DMA,
        ],
    )(mem, idx, val)


def kernel(mem, idx, val):
    return _run(mem, idx.astype(jnp.int32), val)
